# baseline (device time: 25314 ns/iter reference)
import jax
import jax.numpy as jnp
from jax import lax
from jax.experimental import pallas as pl
from jax.experimental.pallas import tpu as pltpu

N_DEV = 4
K = 16
J = 5
GROUPS = 32


def _topk_desc(w, k):
    m = jnp.max(w, axis=1, keepdims=True)
    cols = [m]
    for _ in range(k - 1):
        m = jnp.max(jnp.where(w >= m, -jnp.inf, w), axis=1, keepdims=True)
        cols.append(m)
    return jnp.concatenate(cols, axis=1)


def _insert(r, t):
    for i in range(len(r)):
        hi = jnp.maximum(r[i], t)
        t = jnp.minimum(r[i], t)
        r[i] = hi


def kernel(x):
    m, n = x.shape
    width = n // GROUPS
    n_chunks = 8
    chunk = n // n_chunks
    slabs_per_chunk = chunk // width

    def body(x_hbm, out_ref, comm_ref, x_vmem, copy_sems,
             send_sems, recv_sems):
        my_pos = lax.axis_index("i")

        def chunk_copy(c):
            return pltpu.make_async_copy(
                x_hbm.at[:, pl.ds(c * chunk, chunk)],
                x_vmem.at[c % 2],
                copy_sems.at[c % 2],
            )

        chunk_copy(0).start()

        barrier_sem = pltpu.get_barrier_semaphore()
        for d in range(1, N_DEV):
            pl.semaphore_signal(
                barrier_sem, inc=1,
                device_id=((my_pos + d) % N_DEV,),
                device_id_type=pl.DeviceIdType.MESH,
            )
        pl.semaphore_wait(barrier_sem, N_DEV - 1)

        r = [jnp.full((m, width), -jnp.inf, jnp.float32) for _ in range(J)]
        for c in range(n_chunks):
            chunk_copy(c).wait()
            if c + 1 < n_chunks:
                chunk_copy(c + 1).start()
            for s in range(slabs_per_chunk):
                _insert(r, x_vmem[c % 2, :, s * width:(s + 1) * width])
        comm_ref[0, :, :] = _topk_desc(jnp.concatenate(r, axis=1), K)

        rdmas = []
        for d in range(1, N_DEV):
            rdma = pltpu.make_async_remote_copy(
                src_ref=comm_ref.at[0],
                dst_ref=comm_ref.at[d],
                send_sem=send_sems.at[d],
                recv_sem=recv_sems.at[d],
                device_id=((my_pos + d) % N_DEV,),
                device_id_type=pl.DeviceIdType.MESH,
            )
            rdma.start()
            rdmas.append(rdma)
        for rdma in rdmas:
            rdma.wait()

        cand = jnp.concatenate(
            [comm_ref[i, :, :] for i in range(N_DEV)], axis=1
        )
        out_ref[:, :] = _topk_desc(cand, K)

    return pl.pallas_call(
        body,
        out_shape=jax.ShapeDtypeStruct((m, K), jnp.float32),
        in_specs=[pl.BlockSpec(memory_space=pl.ANY)],
        out_specs=pl.BlockSpec(memory_space=pltpu.VMEM),
        scratch_shapes=[
            pltpu.VMEM((N_DEV, m, K), jnp.float32),
            pltpu.VMEM((2, m, chunk), jnp.float32),
            pltpu.SemaphoreType.DMA((2,)),
            pltpu.SemaphoreType.DMA((N_DEV,)),
            pltpu.SemaphoreType.DMA((N_DEV,)),
        ],
        compiler_params=pltpu.CompilerParams(collective_id=0),
    )(x)
